# baseline (speedup 1.0000x reference)
import jax
import jax.numpy as jnp
from jax.experimental import pallas as pl


def _body(in_ref, slow_ref, fast_ref):
    x = in_ref[...]
    fast_ref[...] = x
    slow_ref[...] = x


def kernel(frames):
    C, T, H, W = frames.shape
    S = T // 4

    grid = (C, T)
    slow, fast = pl.pallas_call(
        _body,
        grid=grid,
        in_specs=[pl.BlockSpec((1, 1, H, W), lambda c, t: (c, t, 0, 0))],
        out_specs=(
            pl.BlockSpec((1, 1, H, W), lambda c, t: (c, (5 * t + 20) // 21, 0, 0)),
            pl.BlockSpec((1, 1, H, W), lambda c, t: (c, t, 0, 0)),
        ),
        out_shape=(
            jax.ShapeDtypeStruct((C, S, H, W), frames.dtype),
            jax.ShapeDtypeStruct((C, T, H, W), frames.dtype),
        ),
    )(frames)
    return (slow, fast)
